# Initial kernel scaffold; baseline (speedup 1.0000x reference)
import jax
import jax.numpy as jnp
from jax.experimental import pallas as pl


def kernel(x, bridge_base_w, bridge_spline_w, conv1_w, conv1_b, conv2_w, conv2_b, pos_emb, W0, a_src0, a_dst0, W1, a_src1, a_dst1, ln_gamma, ln_beta, cls_base_w, cls_spline_w, bridge_grid, cls_grid):
    raise NotImplementedError("write your pallas kernel here")


# speedup vs baseline: 3.3496x; 3.3496x over previous
import functools
import numpy as np
import jax
import jax.numpy as jnp
from jax.experimental import pallas as pl
from jax.experimental.pallas import tpu as pltpu

_GRID_SIZE = 5
_SPLINE_ORDER = 3
_TOPK = 8
_NUM_NODES = 256
_HIDDEN = 128
_IN_DIM = 256
_NUM_CLASSES = 2
_NB = _GRID_SIZE + _SPLINE_ORDER

_H = 2.0 / _GRID_SIZE
_KNOTS = tuple(
    float(np.float32(j * np.float32(_H) - np.float32(1.0)))
    for j in range(-_SPLINE_ORDER, _GRID_SIZE + _SPLINE_ORDER + 1)
)

_SELU_SCALE = 1.0507009873554805
_SELU_ALPHA = 1.6732632423543772


def _bspline_list(x):
    g = _KNOTS
    n0 = len(g) - 1
    bases = [((x >= g[j]) & (x < g[j + 1])).astype(jnp.float32) for j in range(n0)]
    for k in range(1, _SPLINE_ORDER + 1):
        new = []
        for j in range(n0 - k):
            left = (x - g[j]) / (g[j + k] - g[j]) * bases[j]
            right = (g[j + k + 1] - x) / (g[j + k + 1] - g[j + 1]) * bases[j + 1]
            new.append(left + right)
        bases = new
    return bases


def _silu(x):
    return x * jax.nn.sigmoid(x)


def _selu(x):
    return _SELU_SCALE * jnp.where(x > 0, x, _SELU_ALPHA * (jnp.exp(x) - 1.0))


def _elu(x):
    return jnp.where(x > 0, x, jnp.exp(x) - 1.0)


def _bridge_body(x_ref, bw_ref, sw_ref, o_ref):
    x = x_ref[...]
    base = jnp.dot(_silu(x), bw_ref[...], preferred_element_type=jnp.float32)
    bsp = jnp.concatenate(_bspline_list(x), axis=1)
    spline = jnp.dot(bsp, sw_ref[...], preferred_element_type=jnp.float32)
    o_ref[...] = base + spline


def _bridge(x2d, base_w_t, spline_w2):
    n = x2d.shape[0]
    tn = 512
    return pl.pallas_call(
        _bridge_body,
        grid=(n // tn,),
        in_specs=[
            pl.BlockSpec((tn, _IN_DIM), lambda i: (i, 0)),
            pl.BlockSpec((_IN_DIM, _HIDDEN), lambda i: (0, 0)),
            pl.BlockSpec((_NB * _IN_DIM, _HIDDEN), lambda i: (0, 0)),
        ],
        out_specs=pl.BlockSpec((tn, _HIDDEN), lambda i: (i, 0)),
        out_shape=jax.ShapeDtypeStruct((n, _HIDDEN), jnp.float32),
    )(x2d, base_w_t, spline_w2)


def _shift_down(a):
    z = jnp.zeros((1, a.shape[1]), a.dtype)
    return jnp.concatenate([z, a[:-1, :]], axis=0)


def _shift_up(a):
    z = jnp.zeros((1, a.shape[1]), a.dtype)
    return jnp.concatenate([a[1:, :], z], axis=0)


def _conv_body(h_ref, w1_ref, b1_ref, w2_ref, b2_ref, pos_ref, o_ref):
    h = h_ref[0]
    x1 = jnp.concatenate([_shift_down(h), h, _shift_up(h)], axis=1)
    y = jnp.dot(x1, w1_ref[...], preferred_element_type=jnp.float32) + b1_ref[...]
    y = _selu(y)
    x2 = jnp.concatenate([_shift_down(y), y, _shift_up(y)], axis=1)
    y = jnp.dot(x2, w2_ref[...], preferred_element_type=jnp.float32) + b2_ref[...]
    y = _selu(y)
    t = y.shape[0]
    pooled = jnp.mean(y.reshape(_NUM_NODES, t // _NUM_NODES, _HIDDEN), axis=1)
    o_ref[0] = pooled + pos_ref[0]


def _conv_stage(h3d, w1c, b1, w2c, b2, pos_emb):
    b_sz, t, _ = h3d.shape
    return pl.pallas_call(
        _conv_body,
        grid=(b_sz,),
        in_specs=[
            pl.BlockSpec((1, t, _HIDDEN), lambda i: (i, 0, 0)),
            pl.BlockSpec((3 * _HIDDEN, _HIDDEN), lambda i: (0, 0)),
            pl.BlockSpec((1, _HIDDEN), lambda i: (0, 0)),
            pl.BlockSpec((3 * _HIDDEN, _HIDDEN), lambda i: (0, 0)),
            pl.BlockSpec((1, _HIDDEN), lambda i: (0, 0)),
            pl.BlockSpec((1, _NUM_NODES, _HIDDEN), lambda i: (0, 0, 0)),
        ],
        out_specs=pl.BlockSpec((1, _NUM_NODES, _HIDDEN), lambda i: (i, 0, 0)),
        out_shape=jax.ShapeDtypeStruct((b_sz, _NUM_NODES, _HIDDEN), jnp.float32),
    )(h3d, w1c, b1, w2c, b2, pos_emb)


def _kth_threshold(s, axis):
    neg = jnp.float32(-1e30)
    work = s
    removed = jnp.zeros_like(jnp.max(s, axis=axis, keepdims=True))
    kth = jnp.full_like(removed, neg)
    have = jnp.zeros_like(removed, dtype=jnp.bool_)
    for _ in range(_TOPK):
        m = jnp.max(work, axis=axis, keepdims=True)
        c = jnp.sum((work == m).astype(jnp.float32), axis=axis, keepdims=True)
        done_now = jnp.logical_and(removed + c >= _TOPK, jnp.logical_not(have))
        kth = jnp.where(done_now, m, kth)
        have = jnp.logical_or(have, done_now)
        work = jnp.where(work == m, neg, work)
        removed = removed + c
    return kth


def _gat(h, mask, w, a_s, a_d):
    h1 = jnp.dot(h, w, preferred_element_type=jnp.float32)
    src = jnp.dot(h1, a_s, preferred_element_type=jnp.float32)
    dst = jax.lax.dot_general(a_d, h1, (((1,), (1,)), ((), ())),
                              preferred_element_type=jnp.float32)
    e = src + dst
    e = jnp.where(e >= 0, e, 0.2 * e)
    e = jnp.where(mask, e, jnp.float32(-1e9))
    m = jnp.max(e, axis=1, keepdims=True)
    p = jnp.exp(e - m)
    attn = p / jnp.sum(p, axis=1, keepdims=True)
    out = _elu(jnp.dot(attn, h1, preferred_element_type=jnp.float32))
    return h + out


def _head_body(nodes_ref, w0_ref, as0_ref, ad0_ref, w1_ref, as1_ref, ad1_ref,
               g_ref, bta_ref, cbw_ref, csw_ref, emb_ref, log_ref):
    h = nodes_ref[0]
    s = jax.lax.dot_general(h, h, (((1,), (1,)), ((), ())),
                            preferred_element_type=jnp.float32)
    kth_r = _kth_threshold(s, axis=1)
    kth_c = _kth_threshold(s, axis=0)
    n = s.shape[0]
    rows = jax.lax.broadcasted_iota(jnp.int32, (n, n), 0)
    cols = jax.lax.broadcasted_iota(jnp.int32, (n, n), 1)
    mask = (s >= kth_r) | (s >= kth_c) | (rows == cols)

    h = _gat(h, mask, w0_ref[...], as0_ref[...], ad0_ref[...])
    h = _gat(h, mask, w1_ref[...], as1_ref[...], ad1_ref[...])

    mu = jnp.mean(h, axis=1, keepdims=True)
    var = jnp.mean((h - mu) * (h - mu), axis=1, keepdims=True)
    h = (h - mu) * jax.lax.rsqrt(var + 1e-5) * g_ref[...] + bta_ref[...]

    pm = jnp.mean(h, axis=0, keepdims=True)
    px = jnp.max(h, axis=0, keepdims=True)
    emb = jnp.concatenate([pm, px], axis=1)
    emb_ref[0] = emb

    base = jnp.dot(_silu(emb), cbw_ref[...], preferred_element_type=jnp.float32)
    bsp = jnp.concatenate(_bspline_list(emb), axis=1)
    spline = jnp.dot(bsp, csw_ref[...], preferred_element_type=jnp.float32)
    log_ref[0] = base + spline


def _head(nodes, w0, as0, ad0, w1, as1, ad1, g, bta, cbw, csw):
    b_sz = nodes.shape[0]
    emb, log = pl.pallas_call(
        _head_body,
        grid=(b_sz,),
        in_specs=[
            pl.BlockSpec((1, _NUM_NODES, _HIDDEN), lambda i: (i, 0, 0)),
            pl.BlockSpec((_HIDDEN, _HIDDEN), lambda i: (0, 0)),
            pl.BlockSpec((_HIDDEN, 1), lambda i: (0, 0)),
            pl.BlockSpec((1, _HIDDEN), lambda i: (0, 0)),
            pl.BlockSpec((_HIDDEN, _HIDDEN), lambda i: (0, 0)),
            pl.BlockSpec((_HIDDEN, 1), lambda i: (0, 0)),
            pl.BlockSpec((1, _HIDDEN), lambda i: (0, 0)),
            pl.BlockSpec((1, _HIDDEN), lambda i: (0, 0)),
            pl.BlockSpec((1, _HIDDEN), lambda i: (0, 0)),
            pl.BlockSpec((2 * _HIDDEN, _NUM_CLASSES), lambda i: (0, 0)),
            pl.BlockSpec((_NB * 2 * _HIDDEN, _NUM_CLASSES), lambda i: (0, 0)),
        ],
        out_specs=[
            pl.BlockSpec((1, 1, 2 * _HIDDEN), lambda i: (i, 0, 0)),
            pl.BlockSpec((1, 1, _NUM_CLASSES), lambda i: (i, 0, 0)),
        ],
        out_shape=[
            jax.ShapeDtypeStruct((b_sz, 1, 2 * _HIDDEN), jnp.float32),
            jax.ShapeDtypeStruct((b_sz, 1, _NUM_CLASSES), jnp.float32),
        ],
    )(nodes, w0, as0, ad0, w1, as1, ad1, g, bta, cbw, csw)
    return emb.reshape(b_sz, 2 * _HIDDEN), log.reshape(b_sz, _NUM_CLASSES)


@jax.jit
def kernel(x, bridge_base_w, bridge_spline_w, conv1_w, conv1_b, conv2_w,
           conv2_b, pos_emb, W0, a_src0, a_dst0, W1, a_src1, a_dst1,
           ln_gamma, ln_beta, cls_base_w, cls_spline_w, bridge_grid, cls_grid):
    b_sz, t, _ = x.shape

    base_w_t = bridge_base_w.T
    spline_w2 = jnp.transpose(bridge_spline_w, (2, 1, 0)).reshape(_NB * _IN_DIM, _HIDDEN)
    w1c = jnp.transpose(conv1_w, (2, 1, 0)).reshape(3 * _HIDDEN, _HIDDEN)
    w2c = jnp.transpose(conv2_w, (2, 1, 0)).reshape(3 * _HIDDEN, _HIDDEN)
    cbw = cls_base_w.T
    csw = jnp.transpose(cls_spline_w, (2, 1, 0)).reshape(_NB * 2 * _HIDDEN, _NUM_CLASSES)

    h = _bridge(x.reshape(b_sz * t, _IN_DIM), base_w_t, spline_w2)
    nodes = _conv_stage(h.reshape(b_sz, t, _HIDDEN), w1c,
                        conv1_b.reshape(1, _HIDDEN), w2c,
                        conv2_b.reshape(1, _HIDDEN), pos_emb)
    emb, logits = _head(nodes, W0, a_src0.reshape(_HIDDEN, 1),
                        a_dst0.reshape(1, _HIDDEN), W1,
                        a_src1.reshape(_HIDDEN, 1), a_dst1.reshape(1, _HIDDEN),
                        ln_gamma.reshape(1, _HIDDEN), ln_beta.reshape(1, _HIDDEN),
                        cbw, csw)
    return (logits, emb)


# speedup vs baseline: 4.4109x; 1.3168x over previous
import functools
import numpy as np
import jax
import jax.numpy as jnp
from jax.experimental import pallas as pl
from jax.experimental.pallas import tpu as pltpu

_GRID_SIZE = 5
_SPLINE_ORDER = 3
_TOPK = 8
_NUM_NODES = 256
_HIDDEN = 128
_IN_DIM = 256
_NUM_CLASSES = 2
_NB = _GRID_SIZE + _SPLINE_ORDER

_H = 2.0 / _GRID_SIZE
_KNOTS = tuple(
    float(np.float32(j * np.float32(_H) - np.float32(1.0)))
    for j in range(-_SPLINE_ORDER, _GRID_SIZE + _SPLINE_ORDER + 1)
)

_SELU_SCALE = 1.0507009873554805
_SELU_ALPHA = 1.6732632423543772


def _bspline_list(x):
    g0 = _KNOTS[0]
    u = (x - g0) * (1.0 / _H)
    jf = jnp.floor(u)
    t = u - jf
    s = 1.0 - t
    t2 = t * t
    t3 = t2 * t
    w3 = t3 * (1.0 / 6.0)
    w0 = s * s * s * (1.0 / 6.0)
    w2 = (((-3.0 * t + 3.0) * t + 3.0) * t + 1.0) * (1.0 / 6.0)
    w1 = ((3.0 * t - 6.0) * t2 + 4.0) * (1.0 / 6.0)
    E = [(jf == m).astype(jnp.float32) for m in range(_NB + _SPLINE_ORDER)]
    return [w3 * E[c] + w2 * E[c + 1] + w1 * E[c + 2] + w0 * E[c + 3]
            for c in range(_NB)]


def _silu(x):
    return x * jax.nn.sigmoid(x)


def _selu(x):
    return _SELU_SCALE * jnp.where(x > 0, x, _SELU_ALPHA * (jnp.exp(x) - 1.0))


def _elu(x):
    return jnp.where(x > 0, x, jnp.exp(x) - 1.0)


def _bridge_body(x_ref, bw_ref, sw_ref, o_ref):
    x = x_ref[...]
    base = jnp.dot(_silu(x), bw_ref[...], preferred_element_type=jnp.float32)
    bsp = jnp.concatenate(_bspline_list(x), axis=1)
    spline = jnp.dot(bsp, sw_ref[...], preferred_element_type=jnp.float32)
    o_ref[...] = base + spline


def _bridge(x2d, base_w_t, spline_w2):
    n = x2d.shape[0]
    tn = 512
    return pl.pallas_call(
        _bridge_body,
        grid=(n // tn,),
        in_specs=[
            pl.BlockSpec((tn, _IN_DIM), lambda i: (i, 0)),
            pl.BlockSpec((_IN_DIM, _HIDDEN), lambda i: (0, 0)),
            pl.BlockSpec((_NB * _IN_DIM, _HIDDEN), lambda i: (0, 0)),
        ],
        out_specs=pl.BlockSpec((tn, _HIDDEN), lambda i: (i, 0)),
        out_shape=jax.ShapeDtypeStruct((n, _HIDDEN), jnp.float32),
    )(x2d, base_w_t, spline_w2)


def _shift_down(a):
    z = jnp.zeros((1, a.shape[1]), a.dtype)
    return jnp.concatenate([z, a[:-1, :]], axis=0)


def _shift_up(a):
    z = jnp.zeros((1, a.shape[1]), a.dtype)
    return jnp.concatenate([a[1:, :], z], axis=0)


def _conv_body(h_ref, w1_ref, b1_ref, w2_ref, b2_ref, pos_ref, o_ref):
    h = h_ref[0]
    x1 = jnp.concatenate([_shift_down(h), h, _shift_up(h)], axis=1)
    y = jnp.dot(x1, w1_ref[...], preferred_element_type=jnp.float32) + b1_ref[...]
    y = _selu(y)
    x2 = jnp.concatenate([_shift_down(y), y, _shift_up(y)], axis=1)
    y = jnp.dot(x2, w2_ref[...], preferred_element_type=jnp.float32) + b2_ref[...]
    y = _selu(y)
    t = y.shape[0]
    pooled = jnp.mean(y.reshape(_NUM_NODES, t // _NUM_NODES, _HIDDEN), axis=1)
    o_ref[0] = pooled + pos_ref[0]


def _conv_stage(h3d, w1c, b1, w2c, b2, pos_emb):
    b_sz, t, _ = h3d.shape
    return pl.pallas_call(
        _conv_body,
        grid=(b_sz,),
        in_specs=[
            pl.BlockSpec((1, t, _HIDDEN), lambda i: (i, 0, 0)),
            pl.BlockSpec((3 * _HIDDEN, _HIDDEN), lambda i: (0, 0)),
            pl.BlockSpec((1, _HIDDEN), lambda i: (0, 0)),
            pl.BlockSpec((3 * _HIDDEN, _HIDDEN), lambda i: (0, 0)),
            pl.BlockSpec((1, _HIDDEN), lambda i: (0, 0)),
            pl.BlockSpec((1, _NUM_NODES, _HIDDEN), lambda i: (0, 0, 0)),
        ],
        out_specs=pl.BlockSpec((1, _NUM_NODES, _HIDDEN), lambda i: (i, 0, 0)),
        out_shape=jax.ShapeDtypeStruct((b_sz, _NUM_NODES, _HIDDEN), jnp.float32),
    )(h3d, w1c, b1, w2c, b2, pos_emb)


def _kth_threshold(s, axis):
    neg = jnp.float32(-1e30)
    work = s
    removed = jnp.zeros_like(jnp.max(s, axis=axis, keepdims=True))
    kth = jnp.full_like(removed, neg)
    have = jnp.zeros_like(removed, dtype=jnp.bool_)
    for _ in range(_TOPK):
        m = jnp.max(work, axis=axis, keepdims=True)
        c = jnp.sum((work == m).astype(jnp.float32), axis=axis, keepdims=True)
        done_now = jnp.logical_and(removed + c >= _TOPK, jnp.logical_not(have))
        kth = jnp.where(done_now, m, kth)
        have = jnp.logical_or(have, done_now)
        work = jnp.where(work == m, neg, work)
        removed = removed + c
    return kth


def _gat(h, mask, w, a_s, a_d):
    h1 = jnp.dot(h, w, preferred_element_type=jnp.float32)
    src = jnp.dot(h1, a_s, preferred_element_type=jnp.float32)
    dst = jax.lax.dot_general(a_d, h1, (((1,), (1,)), ((), ())),
                              preferred_element_type=jnp.float32)
    e = src + dst
    e = jnp.where(e >= 0, e, 0.2 * e)
    e = jnp.where(mask, e, jnp.float32(-1e9))
    m = jnp.max(e, axis=1, keepdims=True)
    p = jnp.exp(e - m)
    attn = p / jnp.sum(p, axis=1, keepdims=True)
    out = _elu(jnp.dot(attn, h1, preferred_element_type=jnp.float32))
    return h + out


def _head_body(nodes_ref, w0_ref, as0_ref, ad0_ref, w1_ref, as1_ref, ad1_ref,
               g_ref, bta_ref, cbw_ref, csw_ref, emb_ref, log_ref):
    h = nodes_ref[0]
    s = jax.lax.dot_general(h, h, (((1,), (1,)), ((), ())),
                            preferred_element_type=jnp.float32)
    kth_r = _kth_threshold(s, axis=1)
    n = s.shape[0]
    rows = jax.lax.broadcasted_iota(jnp.int32, (n, n), 0)
    cols = jax.lax.broadcasted_iota(jnp.int32, (n, n), 1)
    eye = (rows == cols).astype(jnp.float32)
    a = (s >= kth_r).astype(jnp.float32)
    a_t = jax.lax.dot_general(a, eye, (((0,), (0,)), ((), ())),
                              preferred_element_type=jnp.float32)
    mask = (a + a_t + eye) > 0.5

    h = _gat(h, mask, w0_ref[...], as0_ref[...], ad0_ref[...])
    h = _gat(h, mask, w1_ref[...], as1_ref[...], ad1_ref[...])

    mu = jnp.mean(h, axis=1, keepdims=True)
    var = jnp.mean((h - mu) * (h - mu), axis=1, keepdims=True)
    h = (h - mu) * jax.lax.rsqrt(var + 1e-5) * g_ref[...] + bta_ref[...]

    pm = jnp.mean(h, axis=0, keepdims=True)
    px = jnp.max(h, axis=0, keepdims=True)
    emb = jnp.concatenate([pm, px], axis=1)
    emb_ref[0] = emb

    base = jnp.dot(_silu(emb), cbw_ref[...], preferred_element_type=jnp.float32)
    bsp = jnp.concatenate(_bspline_list(emb), axis=1)
    spline = jnp.dot(bsp, csw_ref[...], preferred_element_type=jnp.float32)
    log_ref[0] = base + spline


def _head(nodes, w0, as0, ad0, w1, as1, ad1, g, bta, cbw, csw):
    b_sz = nodes.shape[0]
    emb, log = pl.pallas_call(
        _head_body,
        grid=(b_sz,),
        in_specs=[
            pl.BlockSpec((1, _NUM_NODES, _HIDDEN), lambda i: (i, 0, 0)),
            pl.BlockSpec((_HIDDEN, _HIDDEN), lambda i: (0, 0)),
            pl.BlockSpec((_HIDDEN, 1), lambda i: (0, 0)),
            pl.BlockSpec((1, _HIDDEN), lambda i: (0, 0)),
            pl.BlockSpec((_HIDDEN, _HIDDEN), lambda i: (0, 0)),
            pl.BlockSpec((_HIDDEN, 1), lambda i: (0, 0)),
            pl.BlockSpec((1, _HIDDEN), lambda i: (0, 0)),
            pl.BlockSpec((1, _HIDDEN), lambda i: (0, 0)),
            pl.BlockSpec((1, _HIDDEN), lambda i: (0, 0)),
            pl.BlockSpec((2 * _HIDDEN, _NUM_CLASSES), lambda i: (0, 0)),
            pl.BlockSpec((_NB * 2 * _HIDDEN, _NUM_CLASSES), lambda i: (0, 0)),
        ],
        out_specs=[
            pl.BlockSpec((1, 1, 2 * _HIDDEN), lambda i: (i, 0, 0)),
            pl.BlockSpec((1, 1, _NUM_CLASSES), lambda i: (i, 0, 0)),
        ],
        out_shape=[
            jax.ShapeDtypeStruct((b_sz, 1, 2 * _HIDDEN), jnp.float32),
            jax.ShapeDtypeStruct((b_sz, 1, _NUM_CLASSES), jnp.float32),
        ],
    )(nodes, w0, as0, ad0, w1, as1, ad1, g, bta, cbw, csw)
    return emb.reshape(b_sz, 2 * _HIDDEN), log.reshape(b_sz, _NUM_CLASSES)


@jax.jit
def kernel(x, bridge_base_w, bridge_spline_w, conv1_w, conv1_b, conv2_w,
           conv2_b, pos_emb, W0, a_src0, a_dst0, W1, a_src1, a_dst1,
           ln_gamma, ln_beta, cls_base_w, cls_spline_w, bridge_grid, cls_grid):
    b_sz, t, _ = x.shape

    base_w_t = bridge_base_w.T
    spline_w2 = jnp.transpose(bridge_spline_w, (2, 1, 0)).reshape(_NB * _IN_DIM, _HIDDEN)
    w1c = jnp.transpose(conv1_w, (2, 1, 0)).reshape(3 * _HIDDEN, _HIDDEN)
    w2c = jnp.transpose(conv2_w, (2, 1, 0)).reshape(3 * _HIDDEN, _HIDDEN)
    cbw = cls_base_w.T
    csw = jnp.transpose(cls_spline_w, (2, 1, 0)).reshape(_NB * 2 * _HIDDEN, _NUM_CLASSES)

    h = _bridge(x.reshape(b_sz * t, _IN_DIM), base_w_t, spline_w2)
    nodes = _conv_stage(h.reshape(b_sz, t, _HIDDEN), w1c,
                        conv1_b.reshape(1, _HIDDEN), w2c,
                        conv2_b.reshape(1, _HIDDEN), pos_emb)
    emb, logits = _head(nodes, W0, a_src0.reshape(_HIDDEN, 1),
                        a_dst0.reshape(1, _HIDDEN), W1,
                        a_src1.reshape(_HIDDEN, 1), a_dst1.reshape(1, _HIDDEN),
                        ln_gamma.reshape(1, _HIDDEN), ln_beta.reshape(1, _HIDDEN),
                        cbw, csw)
    return (logits, emb)
